# Initial kernel scaffold; baseline (speedup 1.0000x reference)
import jax
import jax.numpy as jnp
from jax.experimental import pallas as pl


def kernel(embeddings, target, max_score):
    raise NotImplementedError("write your pallas kernel here")


# speedup vs baseline: 1.2625x; 1.2625x over previous
import jax
import jax.numpy as jnp
from jax.experimental import pallas as pl
from jax.experimental.pallas import tpu as pltpu

_N = 256
_BLK = 8
_MARGIN = 1.0


def _dist_kernel(e_ref, d_ref):
    e = e_ref[...]
    g = jax.lax.dot_general(e, e, (((1,), (1,)), ((), ())),
                            preferred_element_type=jnp.float32)
    sq = jnp.sum(e * e, axis=1, keepdims=True)
    d_ref[...] = sq + jnp.transpose(sq) - 2.0 * g


def _loss_kernel(tgt_sm, drow_ref, tgtr_ref, tgtc_ref,
                 mean_ref, cnt_ref):
    i = pl.program_id(0)

    @pl.when(i == 0)
    def _():
        mean_ref[0, 0] = 0.0
        cnt_ref[0, 0] = 0

    iota_c = jax.lax.broadcasted_iota(jnp.int32, (_N, 1), 0)
    tgtc = tgtc_ref[...]
    tgtr = tgtr_ref[...]
    dcols = jnp.transpose(drow_ref[...])

    total = 0.0
    count = 0
    for j in range(_BLK):
        a = i * _BLK + j
        ta = tgt_sm[a]
        dcol = dcols[:, j:j + 1]
        drow = drow_ref[j:j + 1, :]
        pos = (tgtc == ta) & (iota_c > a)
        neg = tgtr != ta
        m = pos & neg
        l = jnp.maximum(dcol + _MARGIN - drow, 0.0)
        total = total + jnp.sum(jnp.where(m, l, 0.0))
        count = count + jnp.sum(m.astype(jnp.int32))

    mean_ref[0, 0] += total
    cnt_ref[0, 0] += count

    @pl.when(i == pl.num_programs(0) - 1)
    def _():
        c = cnt_ref[0, 0]
        mean_ref[0, 0] = mean_ref[0, 0] / c.astype(jnp.float32)


@jax.jit
def kernel(embeddings, target, max_score):
    dist = pl.pallas_call(
        _dist_kernel,
        out_shape=jax.ShapeDtypeStruct((_N, _N), jnp.float32),
    )(embeddings)

    tgt_r = target.reshape(1, _N)
    tgt_c = target.reshape(_N, 1)

    grid_spec = pltpu.PrefetchScalarGridSpec(
        num_scalar_prefetch=1,
        grid=(_N // _BLK,),
        in_specs=[
            pl.BlockSpec((_BLK, _N), lambda i, *_: (i, 0)),
            pl.BlockSpec((1, _N), lambda i, *_: (0, 0)),
            pl.BlockSpec((_N, 1), lambda i, *_: (0, 0)),
        ],
        out_specs=[
            pl.BlockSpec(memory_space=pltpu.SMEM),
            pl.BlockSpec(memory_space=pltpu.SMEM),
        ],
    )
    mean, cnt = pl.pallas_call(
        _loss_kernel,
        grid_spec=grid_spec,
        out_shape=[
            jax.ShapeDtypeStruct((1, 1), jnp.float32),
            jax.ShapeDtypeStruct((1, 1), jnp.int32),
        ],
    )(target, dist, tgt_r, tgt_c)

    return mean[0, 0], cnt[0, 0]


# speedup vs baseline: 1.6851x; 1.3347x over previous
import jax
import jax.numpy as jnp
from jax.experimental import pallas as pl
from jax.experimental.pallas import tpu as pltpu

_N = 256
_BLK = 8
_MARGIN = 1.0


def _dist_kernel(e_ref, d_ref):
    e = e_ref[...]
    g = jax.lax.dot_general(e, e, (((1,), (1,)), ((), ())),
                            preferred_element_type=jnp.float32)
    sq = jnp.sum(e * e, axis=1, keepdims=True)
    d_ref[...] = sq + jnp.transpose(sq) - 2.0 * g


def _loss_kernel(tgt_sm, drow_ref, tgtr_ref, tgtc_ref,
                 mean_ref, cnt_ref):
    i = pl.program_id(0)

    @pl.when(i == 0)
    def _():
        mean_ref[0, 0] = 0.0
        cnt_ref[0, 0] = 0

    iota_c = jax.lax.broadcasted_iota(jnp.int32, (_N, 1), 0)
    tgtc = tgtc_ref[...]
    tgtr = tgtr_ref[...]
    dcols = jnp.transpose(drow_ref[...])

    big = jnp.float32(1e30)
    accv = jnp.zeros((1, _N), jnp.float32)
    count = 0
    for j in range(_BLK):
        a = i * _BLK + j
        ta = tgt_sm[a]
        pos = (tgtc == ta) & (iota_c > a)
        neg = tgtr != ta
        dcol = jnp.where(pos, dcols[:, j:j + 1] + _MARGIN, -big)
        drow = jnp.where(neg, drow_ref[j:j + 1, :], big)
        l = jnp.maximum(dcol - drow, 0.0)
        accv = accv + jnp.sum(l, axis=0, keepdims=True)
        count = count + (jnp.sum(pos.astype(jnp.int32))
                         * jnp.sum(neg.astype(jnp.int32)))

    mean_ref[0, 0] += jnp.sum(accv)
    cnt_ref[0, 0] += count

    @pl.when(i == pl.num_programs(0) - 1)
    def _():
        c = cnt_ref[0, 0]
        mean_ref[0, 0] = mean_ref[0, 0] / c.astype(jnp.float32)


@jax.jit
def kernel(embeddings, target, max_score):
    dist = pl.pallas_call(
        _dist_kernel,
        out_shape=jax.ShapeDtypeStruct((_N, _N), jnp.float32),
    )(embeddings)

    tgt_r = target.reshape(1, _N)
    tgt_c = target.reshape(_N, 1)

    grid_spec = pltpu.PrefetchScalarGridSpec(
        num_scalar_prefetch=1,
        grid=(_N // _BLK,),
        in_specs=[
            pl.BlockSpec((_BLK, _N), lambda i, *_: (i, 0)),
            pl.BlockSpec((1, _N), lambda i, *_: (0, 0)),
            pl.BlockSpec((_N, 1), lambda i, *_: (0, 0)),
        ],
        out_specs=[
            pl.BlockSpec(memory_space=pltpu.SMEM),
            pl.BlockSpec(memory_space=pltpu.SMEM),
        ],
    )
    mean, cnt = pl.pallas_call(
        _loss_kernel,
        grid_spec=grid_spec,
        out_shape=[
            jax.ShapeDtypeStruct((1, 1), jnp.float32),
            jax.ShapeDtypeStruct((1, 1), jnp.int32),
        ],
    )(target, dist, tgt_r, tgt_c)

    return mean[0, 0], cnt[0, 0]


# speedup vs baseline: 1.9786x; 1.1742x over previous
import dataclasses
import functools

import jax
import jax.numpy as jnp
from jax import lax
from jax.experimental import pallas as pl
from jax.experimental.pallas import tpu as pltpu
from jax.experimental.pallas import tpu_sc as plsc

_N = 256
_MARGIN = 1.0
_NSUB = 16
_L = 16


def _dist_kernel(e_ref, d_ref):
    e = e_ref[...]
    g = jax.lax.dot_general(e, e, (((1,), (1,)), ((), ())),
                            preferred_element_type=jnp.float32)
    sq = jnp.sum(e * e, axis=1, keepdims=True)
    d_ref[...] = sq + jnp.transpose(sq) - 2.0 * g


def _sc_body(dist_hbm, tgt_hbm, mean_hbm, cnt_hbm, part_hbm,
             drow_v, tgt_v, plist_v, fbuf_v, cbuf_v, ibuf_v, tmpf_v):
    cid = lax.axis_index("c")
    sid = lax.axis_index("s")

    @pl.when(cid == 0)
    def _compute():
        s8 = sid * 8
        pltpu.sync_copy(tgt_hbm, tgt_v)
        pltpu.sync_copy(dist_hbm.at[pl.ds(s8, 8)], drow_v.at[pl.ds(0, 8)])
        pltpu.sync_copy(dist_hbm.at[pl.ds(248 - s8, 8)],
                        drow_v.at[pl.ds(8, 8)])

        lanes = lax.iota(jnp.int32, _L)
        big = jnp.float32(1e30)
        accv = jnp.zeros((_L,), jnp.float32)
        cntv = jnp.zeros((_L,), jnp.int32)

        for j in range(16):
            if j < 8:
                a = s8 + j
            else:
                a = 248 - s8 + (j - 8)
            ta_vec = plsc.load_gather(tgt_v, [jnp.full((_L,), a, jnp.int32)])

            npos_vec = jnp.zeros((_L,), jnp.int32)
            nneg_vec = jnp.zeros((_L,), jnp.int32)
            dms = []
            for c in range(_N // _L):
                tgtc = tgt_v[pl.ds(c * _L, _L)]
                drc = drow_v[j, pl.ds(c * _L, _L)]
                pidx = lanes + (c * _L)
                posm = (tgtc == ta_vec) & (pidx > a)
                negm = tgtc != ta_vec
                dms.append(jnp.where(negm, drc, big))
                cpos = plsc.cumsum(posm.astype(jnp.int32))
                plsc.store_scatter(plist_v, [npos_vec + cpos - 1], pidx,
                                   mask=posm)
                npos_vec = npos_vec + plsc.all_reduce_population_count(posm)
                nneg_vec = nneg_vec + plsc.all_reduce_population_count(negm)

            cntv = cntv + npos_vec * nneg_vec
            npos_s = jnp.max(npos_vec)

            def pair_body(i, acc, j=j, dms=dms):
                pvec = plsc.load_gather(plist_v,
                                        [jnp.full((_L,), i, jnp.int32)])
                t = plsc.load_gather(
                    drow_v, [jnp.full((_L,), j, jnp.int32), pvec])
                t = t + jnp.float32(_MARGIN)
                for c in range(_N // _L):
                    acc = acc + jnp.maximum(t - dms[c], 0.0)
                return acc

            accv = lax.fori_loop(0, npos_s, pair_body, accv)

        fbuf_v[...] = accv
        cbuf_v[...] = jnp.where(lanes == 0, cntv, 0).astype(jnp.float32)
        pltpu.sync_copy(fbuf_v, part_hbm.at[sid])
        pltpu.sync_copy(cbuf_v, part_hbm.at[_NSUB + sid])
        plsc.subcore_barrier()

        @pl.when(sid == 0)
        def _finalize():
            pltpu.sync_copy(part_hbm, tmpf_v)
            tf = jnp.zeros((_L,), jnp.float32)
            tc = jnp.zeros((_L,), jnp.float32)
            for r in range(_NSUB):
                tf = tf + tmpf_v[r, pl.ds(0, _L)]
                tc = tc + tmpf_v[_NSUB + r, pl.ds(0, _L)]
            total = jnp.sum(tf)
            cnt_s = jnp.sum(tc)
            totv = jnp.full((_L,), total)
            cntfv = jnp.full((_L,), cnt_s)
            fbuf_v[...] = totv / cntfv
            ibuf_v[...] = cntfv.astype(jnp.int32)
            pltpu.sync_copy(fbuf_v, mean_hbm)
            pltpu.sync_copy(ibuf_v, cnt_hbm)


@jax.jit
def kernel(embeddings, target, max_score):
    dist = pl.pallas_call(
        _dist_kernel,
        out_shape=jax.ShapeDtypeStruct((_N, _N), jnp.float32),
    )(embeddings)

    mesh = plsc.VectorSubcoreMesh(core_axis_name="c", subcore_axis_name="s")
    cp = pltpu.CompilerParams()
    if "needs_layout_passes" in pltpu.CompilerParams.__dataclass_fields__:
        cp = dataclasses.replace(cp, needs_layout_passes=False)
    sc_loss = functools.partial(
        pl.kernel,
        mesh=mesh,
        compiler_params=cp,
        out_type=[
            jax.ShapeDtypeStruct((_L,), jnp.float32),
            jax.ShapeDtypeStruct((_L,), jnp.int32),
            jax.ShapeDtypeStruct((2 * _NSUB, _L), jnp.float32),
        ],
        scratch_types=[
            pltpu.VMEM((16, _N), jnp.float32),
            pltpu.VMEM((_N,), jnp.int32),
            pltpu.VMEM((_N,), jnp.int32),
            pltpu.VMEM((_L,), jnp.float32),
            pltpu.VMEM((_L,), jnp.float32),
            pltpu.VMEM((_L,), jnp.int32),
            pltpu.VMEM((2 * _NSUB, _L), jnp.float32),
        ],
    )(_sc_body)
    mean16, cnt16, _ = sc_loss(dist, target)
    return mean16[0], cnt16[0]


# speedup vs baseline: 2.3602x; 1.1929x over previous
import dataclasses
import functools

import jax
import jax.numpy as jnp
from jax import lax
from jax.experimental import pallas as pl
from jax.experimental.pallas import tpu as pltpu
from jax.experimental.pallas import tpu_sc as plsc

_N = 256
_MARGIN = 1.0
_L = 16
_NW = 32


def _dist_kernel(e_ref, d_ref):
    e = e_ref[...]
    g = jax.lax.dot_general(e, e, (((1,), (1,)), ((), ())),
                            preferred_element_type=jnp.float32)
    sq = jnp.sum(e * e, axis=1, keepdims=True)
    d_ref[...] = sq + jnp.transpose(sq) - 2.0 * g


def _sc_body(dist_hbm, tgt_hbm, part_hbm, drow_v, tgt_v, plist_v,
             fbuf_v, cbuf_v):
    cid = lax.axis_index("c")
    sid = lax.axis_index("s")
    w = sid * 2 + cid
    w4 = w * 4

    pltpu.sync_copy(tgt_hbm, tgt_v)
    pltpu.sync_copy(dist_hbm.at[pl.ds(w4, 4)], drow_v.at[pl.ds(0, 4)])
    pltpu.sync_copy(dist_hbm.at[pl.ds(252 - w4, 4)], drow_v.at[pl.ds(4, 4)])

    lanes = lax.iota(jnp.int32, _L)
    big = jnp.float32(1e30)
    accv = jnp.zeros((_L,), jnp.float32)
    cntv = jnp.zeros((_L,), jnp.int32)

    for j in range(8):
        if j < 4:
            a = w4 + j
        else:
            a = 252 - w4 + (j - 4)
        ta_vec = plsc.load_gather(tgt_v, [jnp.full((_L,), a, jnp.int32)])

        npos_vec = jnp.zeros((_L,), jnp.int32)
        nneg_vec = jnp.zeros((_L,), jnp.int32)
        dms = []
        for c in range(_N // _L):
            tgtc = tgt_v[pl.ds(c * _L, _L)]
            drc = drow_v[j, pl.ds(c * _L, _L)]
            pidx = lanes + (c * _L)
            posm = (tgtc == ta_vec) & (pidx > a)
            negm = tgtc != ta_vec
            dms.append(jnp.where(negm, drc, big))
            cpos = plsc.cumsum(posm.astype(jnp.int32))
            plsc.store_scatter(plist_v, [npos_vec + cpos - 1], pidx,
                               mask=posm)
            npos_vec = npos_vec + plsc.all_reduce_population_count(posm)
            nneg_vec = nneg_vec + plsc.all_reduce_population_count(negm)

        cntv = cntv + npos_vec * nneg_vec
        npos_s = jnp.max(npos_vec)

        def pair_body(i, acc, j=j, dms=dms):
            pvec = plsc.load_gather(plist_v, [jnp.full((_L,), i, jnp.int32)])
            t = plsc.load_gather(drow_v, [jnp.full((_L,), j, jnp.int32), pvec])
            t = t + jnp.float32(_MARGIN)
            for c in range(_N // _L):
                acc = acc + jnp.maximum(t - dms[c], 0.0)
            return acc

        accv = lax.fori_loop(0, npos_s, pair_body, accv)

    fbuf_v[...] = accv
    cbuf_v[...] = jnp.where(lanes == 0, cntv, 0).astype(jnp.float32)
    pltpu.sync_copy(fbuf_v, part_hbm.at[w])
    pltpu.sync_copy(cbuf_v, part_hbm.at[_NW + w])


def _fin_kernel(part_ref, mean_ref, cnt_ref):
    p = part_ref[...]
    total = jnp.sum(p[:_NW, :])
    cnt = jnp.sum(p[_NW:, :])
    mean_ref[0, 0] = total / cnt
    cnt_ref[0, 0] = cnt.astype(jnp.int32)


@jax.jit
def kernel(embeddings, target, max_score):
    dist = pl.pallas_call(
        _dist_kernel,
        out_shape=jax.ShapeDtypeStruct((_N, _N), jnp.float32),
    )(embeddings)

    mesh = plsc.VectorSubcoreMesh(core_axis_name="c", subcore_axis_name="s")
    cp = pltpu.CompilerParams()
    if "needs_layout_passes" in pltpu.CompilerParams.__dataclass_fields__:
        cp = dataclasses.replace(cp, needs_layout_passes=False)
    sc_loss = functools.partial(
        pl.kernel,
        mesh=mesh,
        compiler_params=cp,
        out_type=jax.ShapeDtypeStruct((2 * _NW, _L), jnp.float32),
        scratch_types=[
            pltpu.VMEM((8, _N), jnp.float32),
            pltpu.VMEM((_N,), jnp.int32),
            pltpu.VMEM((_N,), jnp.int32),
            pltpu.VMEM((_L,), jnp.float32),
            pltpu.VMEM((_L,), jnp.float32),
        ],
    )(_sc_body)
    part = sc_loss(dist, target)

    mean, cnt = pl.pallas_call(
        _fin_kernel,
        out_shape=[
            jax.ShapeDtypeStruct((1, 1), jnp.float32),
            jax.ShapeDtypeStruct((1, 1), jnp.int32),
        ],
        out_specs=[
            pl.BlockSpec(memory_space=pltpu.SMEM),
            pl.BlockSpec(memory_space=pltpu.SMEM),
        ],
    )(part)

    return mean[0, 0], cnt[0, 0]


# speedup vs baseline: 2.6607x; 1.1273x over previous
import dataclasses
import functools

import jax
import jax.numpy as jnp
from jax import lax
from jax.experimental import pallas as pl
from jax.experimental.pallas import tpu as pltpu
from jax.experimental.pallas import tpu_sc as plsc

_N = 256
_MARGIN = 1.0
_L = 16
_NW = 32


def _dist_kernel(e_ref, d_ref):
    e = e_ref[...]
    g = jax.lax.dot_general(e, e, (((1,), (1,)), ((), ())),
                            preferred_element_type=jnp.float32)
    sq = jnp.sum(e * e, axis=1, keepdims=True)
    d_ref[...] = sq + jnp.transpose(sq) - 2.0 * g


def _sc_body(dist_hbm, tgt_hbm, part_hbm, drow_v, tgt_v, plist_v,
             obuf_v, sem):
    cid = lax.axis_index("c")
    sid = lax.axis_index("s")
    w = sid * 2 + cid
    w4 = w * 4

    c1 = pltpu.async_copy(tgt_hbm, tgt_v, sem)
    c2 = pltpu.async_copy(dist_hbm.at[pl.ds(w4, 4)],
                          drow_v.at[pl.ds(0, 4)], sem)
    c3 = pltpu.async_copy(dist_hbm.at[pl.ds(252 - w4, 4)],
                          drow_v.at[pl.ds(4, 4)], sem)
    c1.wait()
    c2.wait()
    c3.wait()

    lanes = lax.iota(jnp.int32, _L)
    big = jnp.float32(1e30)
    accv = jnp.zeros((_L,), jnp.float32)
    cntv = jnp.zeros((_L,), jnp.int32)
    tchunks = [tgt_v[pl.ds(c * _L, _L)] for c in range(_N // _L)]

    for j in range(8):
        if j < 4:
            a = w4 + j
        else:
            a = 252 - w4 + (j - 4)
        ta_vec = plsc.load_gather(tgt_v, [jnp.full((_L,), a, jnp.int32)])

        npos_vec = jnp.zeros((_L,), jnp.int32)
        nneg_vec = jnp.zeros((_L,), jnp.int32)
        dms = []
        for c in range(_N // _L):
            tgtc = tchunks[c]
            drc = drow_v[j, pl.ds(c * _L, _L)]
            negm = tgtc != ta_vec
            dms.append(jnp.where(negm, drc, big))
            nneg_vec = nneg_vec + plsc.all_reduce_population_count(negm)
            if j >= 4 and c < 8:
                continue
            pidx = lanes + (c * _L)
            posm = (tgtc == ta_vec) & (pidx > a)
            cpos = plsc.cumsum(posm.astype(jnp.int32))
            plsc.store_scatter(plist_v, [npos_vec + cpos - 1], pidx,
                               mask=posm)
            npos_vec = npos_vec + plsc.all_reduce_population_count(posm)

        cntv = cntv + npos_vec * nneg_vec
        npos_s = jnp.max(npos_vec)

        def pair_body(i, acc, j=j, dms=dms):
            pvec = plsc.load_gather(plist_v, [jnp.full((_L,), i, jnp.int32)])
            t = plsc.load_gather(drow_v, [jnp.full((_L,), j, jnp.int32), pvec])
            t = t + jnp.float32(_MARGIN)
            for c in range(_N // _L):
                acc = acc + jnp.maximum(t - dms[c], 0.0)
            return acc

        accv = lax.fori_loop(0, npos_s, pair_body, accv)

    obuf_v[pl.ds(0, _L)] = accv
    obuf_v[pl.ds(_L, _L)] = jnp.where(lanes == 0, cntv, 0).astype(jnp.float32)
    pltpu.sync_copy(obuf_v, part_hbm.at[w])


def _fin_kernel(part_ref, mean_ref, cnt_ref):
    p = part_ref[...]
    total = jnp.sum(p[:, :_L])
    cnt = jnp.sum(p[:, _L:])
    mean_ref[0, 0] = total / cnt
    cnt_ref[0, 0] = cnt.astype(jnp.int32)


@jax.jit
def kernel(embeddings, target, max_score):
    dist = pl.pallas_call(
        _dist_kernel,
        out_shape=jax.ShapeDtypeStruct((_N, _N), jnp.float32),
    )(embeddings)

    mesh = plsc.VectorSubcoreMesh(core_axis_name="c", subcore_axis_name="s")
    cp = pltpu.CompilerParams()
    if "needs_layout_passes" in pltpu.CompilerParams.__dataclass_fields__:
        cp = dataclasses.replace(cp, needs_layout_passes=False)
    sc_loss = functools.partial(
        pl.kernel,
        mesh=mesh,
        compiler_params=cp,
        out_type=jax.ShapeDtypeStruct((_NW, 2 * _L), jnp.float32),
        scratch_types=[
            pltpu.VMEM((8, _N), jnp.float32),
            pltpu.VMEM((_N,), jnp.int32),
            pltpu.VMEM((_N,), jnp.int32),
            pltpu.VMEM((2 * _L,), jnp.float32),
            pltpu.SemaphoreType.DMA,
        ],
    )(_sc_body)
    part = sc_loss(dist, target)

    mean, cnt = pl.pallas_call(
        _fin_kernel,
        out_shape=[
            jax.ShapeDtypeStruct((1, 1), jnp.float32),
            jax.ShapeDtypeStruct((1, 1), jnp.int32),
        ],
        out_specs=[
            pl.BlockSpec(memory_space=pltpu.SMEM),
            pl.BlockSpec(memory_space=pltpu.SMEM),
        ],
    )(part)

    return mean[0, 0], cnt[0, 0]
